# baseline (speedup 1.0000x reference)
import functools

import jax
import jax.numpy as jnp
from jax import lax
from jax.experimental import pallas as pl
from jax.experimental.pallas import tpu as pltpu
from jax.experimental.pallas import tpu_sc as plsc

_SCENE = (256, 256, 32)
_PS = 2
_SX, _SY, _SZ = _SCENE[0] // _PS, _SCENE[1] // _PS, _SCENE[2] // _PS

_C, _H, _W = 128, 96, 320
_HW = _H * _W
_N = _SX * _SY * _SZ
_XS = _SY * _SZ

_NWORKERS = 32
_SL = _N // _NWORKERS
_CPW = _C // _NWORKERS
_PPW = _CPW // 2
_CH = 2048
_NCH = _N // _CH

_mesh = plsc.VectorSubcoreMesh(core_axis_name="c", subcore_axis_name="s")
_params = pltpu.CompilerParams(needs_layout_passes=False,
                               use_tc_tiling_on_sc=False)


@functools.partial(
    pl.kernel,
    out_type=jax.ShapeDtypeStruct((_N,), jnp.int32),
    mesh=_mesh,
    compiler_params=_params,
    scratch_types=[
        pltpu.VMEM((_HW,), jnp.float32),
        pltpu.VMEM((_SL,), jnp.int32),
        pltpu.VMEM((_SL,), jnp.int32),
        pltpu.VMEM((16,), jnp.float32),
        pltpu.VMEM((_SL,), jnp.float32),
        pltpu.VMEM((_SL,), jnp.float32),
        pltpu.VMEM((_SL,), jnp.int32),
    ],
)
def _idx_weight(xs_hbm, ys_hbm, rcp_hbm, fov_hbm, pz_hbm, depth_hbm,
                pk_hbm,
                depth_v, xs_v, ys_v, rcp_v, fov_v, pz_v, pk_v):
    wid = lax.axis_index("s") * 2 + lax.axis_index("c")
    base = wid * _SL
    pltpu.sync_copy(depth_hbm, depth_v)
    pltpu.sync_copy(xs_hbm.at[pl.ds(base, _SL)], xs_v)
    pltpu.sync_copy(ys_hbm.at[pl.ds(base, _SL)], ys_v)
    pltpu.sync_copy(rcp_hbm, rcp_v)
    pltpu.sync_copy(fov_hbm.at[pl.ds(base, _SL)], fov_v)
    pltpu.sync_copy(pz_hbm.at[pl.ds(base, _SL)], pz_v)

    rcp = rcp_v[...]
    iota16 = lax.iota(jnp.int32, 16)

    @plsc.parallel_loop(0, _SL // 16, unroll=2)
    def _(j):
        m = j * 16
        nv = (j // 128) * _XS + ((j % 8) * 16 + iota16) * _SZ + (j // 8) % 16
        x = plsc.load_gather(xs_v, [nv])
        y = plsc.load_gather(ys_v, [nv])
        di = y * _W + x
        xs_ = ((x.astype(jnp.float32) + 0.5) * rcp).astype(jnp.int32)
        ys_ = ((y.astype(jnp.float32) + 0.5) * rcp).astype(jnp.int32)
        d = plsc.load_gather(depth_v, [di])
        t = plsc.load_gather(pz_v, [nv]) - d
        wgt = jnp.exp(t * t * -2.0)
        wgt = jnp.where(d == 0.0, jnp.float32(1.0), wgt)
        wgt = wgt * plsc.load_gather(fov_v, [nv])
        wb = plsc.bitcast(wgt, jnp.int32)
        wb = (wb + 0x8000) & jnp.int32(-65536)
        pk_v[pl.ds(m, 16)] = wb | (ys_ * _W + xs_)

    pltpu.sync_copy(pk_v, pk_hbm.at[pl.ds(base, _SL)])


@functools.partial(
    pl.kernel,
    out_type=jax.ShapeDtypeStruct((_C, _N), jnp.float32),
    mesh=_mesh,
    compiler_params=_params,
    scratch_types=[
        pltpu.VMEM((_HW,), jnp.int32),
        pltpu.VMEM((_HW,), jnp.int32),
        pltpu.VMEM((2, _CH), jnp.int32),
        pltpu.VMEM((2, _CH), jnp.float32),
        pltpu.VMEM((2, _CH), jnp.float32),
        pltpu.VMEM((2, _CH), jnp.float32),
        pltpu.VMEM((2, _CH), jnp.float32),
        pltpu.SemaphoreType.DMA,
        pltpu.SemaphoreType.DMA,
        pltpu.SemaphoreType.DMA,
        pltpu.SemaphoreType.DMA,
    ],
)
def _gather_scale(src_hbm, pk_hbm, out_hbm,
                  r0, r1, pk2, ol0, ol1, oh0, oh1,
                  sin0, sin1, sout0, sout1):
    wid = lax.axis_index("s") * 2 + lax.axis_index("c")
    p0 = wid * _PPW
    rows = (r0, r1)
    outs = (ol0, ol1, oh0, oh1)
    ochan = (p0, p0 + 1, 64 + p0, 64 + p0 + 1)
    sins = (sin0, sin1)
    souts = (sout0, sout1)

    def start_in(k, b):
        pltpu.async_copy(pk_hbm.at[pl.ds(k * _CH, _CH)], pk2.at[b], sins[b])

    def wait_in(k, b):
        pltpu.make_async_copy(pk_hbm.at[pl.ds(k * _CH, _CH)], pk2.at[b],
                              sins[b]).wait()

    def out_copies(k, b):
        return [
            pltpu.make_async_copy(
                outs[c].at[b],
                out_hbm.at[ochan[c], pl.ds(k * _CH, _CH)],
                souts[b])
            for c in range(4)
        ]

    start_in(0, 0)
    start_in(1, 1)
    for c in range(_PPW):
        pltpu.sync_copy(src_hbm.at[p0 + c], rows[c])

    def step(i, carry):
        for b in range(2):
            k = 2 * i + b
            wait_in(k, b)

            @pl.when(i >= 1)
            def _():
                for cp in out_copies(k - 2, b):
                    cp.wait()

            @plsc.parallel_loop(0, _CH // 16, unroll=16)
            def _(j):
                o = j * 16
                pk = pk2[b, pl.ds(o, 16)]
                iv = pk & 0xFFFF
                wv = plsc.bitcast(pk & jnp.int32(-65536), jnp.float32)
                for c in range(_PPW):
                    g = plsc.load_gather(rows[c], [iv])
                    glo = plsc.bitcast(lax.shift_left(g, 16), jnp.float32)
                    ghi = plsc.bitcast(g & jnp.int32(-65536), jnp.float32)
                    outs[c][b, pl.ds(o, 16)] = glo * wv
                    outs[2 + c][b, pl.ds(o, 16)] = ghi * wv

            for cp in out_copies(k, b):
                cp.start()

            @pl.when(i < _NCH // 2 - 1)
            def _():
                start_in(k + 2, b)
        return carry

    lax.fori_loop(0, _NCH // 2, step, 0)
    for b in range(2):
        for cp in out_copies(_NCH - 2 + b, b):
            cp.wait()


def kernel(x2d, projected_pix, scale_2d, fov_mask, pix_z, depth_img):
    c, h, w = x2d.shape
    xs = projected_pix[:, 0]
    ys = projected_pix[:, 1]
    rcp_vec = jnp.full((16,), 1.0, jnp.float32) / jnp.float32(scale_2d)
    fov_f = fov_mask.astype(jnp.float32)
    pz = pix_z.reshape(-1)
    depth_flat = depth_img.reshape(-1)
    bf = x2d.reshape(c, h * w).astype(jnp.bfloat16)
    src_pk = lax.bitcast_convert_type(
        jnp.stack([bf[: c // 2], bf[c // 2:]], axis=-1), jnp.int32)

    pk = _idx_weight(xs, ys, rcp_vec, fov_f, pz, depth_flat)
    out = _gather_scale(src_pk, pk)
    return jnp.transpose(out.reshape(c, _SX, _SZ, _SY), (0, 1, 3, 2))


# speedup vs baseline: 1.0027x; 1.0027x over previous
import functools

import jax
import jax.numpy as jnp
from jax import lax
from jax.experimental import pallas as pl
from jax.experimental.pallas import tpu as pltpu
from jax.experimental.pallas import tpu_sc as plsc

_SCENE = (256, 256, 32)
_PS = 2
_SX, _SY, _SZ = _SCENE[0] // _PS, _SCENE[1] // _PS, _SCENE[2] // _PS

_C, _H, _W = 128, 96, 320
_HW = _H * _W
_N = _SX * _SY * _SZ
_XS = _SY * _SZ

_NWORKERS = 32
_SL = _N // _NWORKERS
_CPW = _C // _NWORKERS
_PPW = _CPW // 2
_CH = 2048
_NCH = _N // _CH

_mesh = plsc.VectorSubcoreMesh(core_axis_name="c", subcore_axis_name="s")
_params = pltpu.CompilerParams(needs_layout_passes=False,
                               use_tc_tiling_on_sc=False)


@functools.partial(
    pl.kernel,
    out_type=jax.ShapeDtypeStruct((_N,), jnp.int32),
    mesh=_mesh,
    compiler_params=_params,
    scratch_types=[
        pltpu.VMEM((_HW,), jnp.float32),
        pltpu.VMEM((_SL,), jnp.int32),
        pltpu.VMEM((_SL,), jnp.int32),
        pltpu.VMEM((16,), jnp.float32),
        pltpu.VMEM((_SL,), jnp.float32),
        pltpu.VMEM((_SL,), jnp.float32),
        pltpu.VMEM((_SL,), jnp.int32),
    ],
)
def _idx_weight(xs_hbm, ys_hbm, rcp_hbm, fov_hbm, pz_hbm, depth_hbm,
                pk_hbm,
                depth_v, xs_v, ys_v, rcp_v, fov_v, pz_v, pk_v):
    wid = lax.axis_index("s") * 2 + lax.axis_index("c")
    base = wid * _SL
    pltpu.sync_copy(depth_hbm, depth_v)
    pltpu.sync_copy(xs_hbm.at[pl.ds(base, _SL)], xs_v)
    pltpu.sync_copy(ys_hbm.at[pl.ds(base, _SL)], ys_v)
    pltpu.sync_copy(rcp_hbm, rcp_v)
    pltpu.sync_copy(fov_hbm.at[pl.ds(base, _SL)], fov_v)
    pltpu.sync_copy(pz_hbm.at[pl.ds(base, _SL)], pz_v)

    rcp = rcp_v[...]
    iota16 = lax.iota(jnp.int32, 16)

    @plsc.parallel_loop(0, _SL // 16, unroll=2)
    def _(j):
        m = j * 16
        nv = (j // 128) * _XS + ((j % 8) * 16 + iota16) * _SZ + (j // 8) % 16
        x = plsc.load_gather(xs_v, [nv])
        y = plsc.load_gather(ys_v, [nv])
        di = y * _W + x
        xs_ = ((x.astype(jnp.float32) + 0.5) * rcp).astype(jnp.int32)
        ys_ = ((y.astype(jnp.float32) + 0.5) * rcp).astype(jnp.int32)
        d = plsc.load_gather(depth_v, [di])
        t = plsc.load_gather(pz_v, [nv]) - d
        wgt = jnp.exp(t * t * -2.0)
        wgt = jnp.where(d == 0.0, jnp.float32(1.0), wgt)
        wgt = wgt * plsc.load_gather(fov_v, [nv])
        wb = plsc.bitcast(wgt, jnp.int32)
        wb = (wb + 0x8000) & jnp.int32(-65536)
        pk_v[pl.ds(m, 16)] = wb | (ys_ * _W + xs_)

    pltpu.sync_copy(pk_v, pk_hbm.at[pl.ds(base, _SL)])


@functools.partial(
    pl.kernel,
    out_type=jax.ShapeDtypeStruct((_C, _N), jnp.float32),
    mesh=_mesh,
    compiler_params=_params,
    scratch_types=[
        pltpu.VMEM((_HW,), jnp.int32),
        pltpu.VMEM((_HW,), jnp.int32),
        pltpu.VMEM((2, _CH), jnp.int32),
        pltpu.VMEM((2, _CH), jnp.float32),
        pltpu.VMEM((2, _CH), jnp.float32),
        pltpu.VMEM((2, _CH), jnp.float32),
        pltpu.VMEM((2, _CH), jnp.float32),
        pltpu.SemaphoreType.DMA,
        pltpu.SemaphoreType.DMA,
        pltpu.SemaphoreType.DMA,
        pltpu.SemaphoreType.DMA,
    ],
)
def _gather_scale(src_hbm, pk_hbm, out_hbm,
                  r0, r1, pk2, ol0, ol1, oh0, oh1,
                  sin0, sin1, sout0, sout1):
    wid = lax.axis_index("s") * 2 + lax.axis_index("c")
    p0 = wid * _PPW
    rows = (r0, r1)
    outs = (ol0, ol1, oh0, oh1)
    ochan = (p0, p0 + 1, 64 + p0, 64 + p0 + 1)
    sins = (sin0, sin1)
    souts = (sout0, sout1)

    def start_in(k, b):
        pltpu.async_copy(pk_hbm.at[pl.ds(k * _CH, _CH)], pk2.at[b], sins[b])

    def wait_in(k, b):
        pltpu.make_async_copy(pk_hbm.at[pl.ds(k * _CH, _CH)], pk2.at[b],
                              sins[b]).wait()

    def out_copies(k, b):
        return [
            pltpu.make_async_copy(
                outs[c].at[b],
                out_hbm.at[ochan[c], pl.ds(k * _CH, _CH)],
                souts[b])
            for c in range(4)
        ]

    start_in(0, 0)
    start_in(1, 1)
    for c in range(_PPW):
        pltpu.sync_copy(src_hbm.at[p0 + c], rows[c])

    def step(i, carry):
        for b in range(2):
            k = 2 * i + b
            wait_in(k, b)

            @pl.when(i >= 1)
            def _():
                for cp in out_copies(k - 2, b):
                    cp.wait()

            @plsc.parallel_loop(0, _CH // 16, unroll=8)
            def _(j):
                o = j * 16
                pk = pk2[b, pl.ds(o, 16)]
                iv = pk & 0xFFFF
                wv = plsc.bitcast(pk & jnp.int32(-65536), jnp.float32)
                for c in range(_PPW):
                    g = plsc.load_gather(rows[c], [iv])
                    glo = plsc.bitcast(lax.shift_left(g, 16), jnp.float32)
                    ghi = plsc.bitcast(g & jnp.int32(-65536), jnp.float32)
                    outs[c][b, pl.ds(o, 16)] = glo * wv
                    outs[2 + c][b, pl.ds(o, 16)] = ghi * wv

            for cp in out_copies(k, b):
                cp.start()

            @pl.when(i < _NCH // 2 - 1)
            def _():
                start_in(k + 2, b)
        return carry

    lax.fori_loop(0, _NCH // 2, step, 0)
    for b in range(2):
        for cp in out_copies(_NCH - 2 + b, b):
            cp.wait()


def kernel(x2d, projected_pix, scale_2d, fov_mask, pix_z, depth_img):
    c, h, w = x2d.shape
    xs = projected_pix[:, 0]
    ys = projected_pix[:, 1]
    rcp_vec = jnp.full((16,), 1.0, jnp.float32) / jnp.float32(scale_2d)
    fov_f = fov_mask.astype(jnp.float32)
    pz = pix_z.reshape(-1)
    depth_flat = depth_img.reshape(-1)
    bf = x2d.reshape(c, h * w).astype(jnp.bfloat16)
    src_pk = lax.bitcast_convert_type(
        jnp.stack([bf[: c // 2], bf[c // 2:]], axis=-1), jnp.int32)

    pk = _idx_weight(xs, ys, rcp_vec, fov_f, pz, depth_flat)
    out = _gather_scale(src_pk, pk)
    return jnp.transpose(out.reshape(c, _SX, _SZ, _SY), (0, 1, 3, 2))
